# Initial kernel scaffold; baseline (speedup 1.0000x reference)
import jax
import jax.numpy as jnp
from jax.experimental import pallas as pl


def kernel(x, edge_index, edge_attr, batch, lin1_W, lin1_b, att_l, att_r, gate_lin1_W, gate_lin2_W, gate_bias, gru0_Wi, gru0_Wh, gru0_bi, gru0_bh, gat_W, gat_att_src, gat_att_dst, gat_bias, gru1_Wi, gru1_Wh, gru1_bi, gru1_bh, mol_W, mol_att_src, mol_att_dst, mol_bias, mgru_Wi, mgru_Wh, mgru_bi, mgru_bh, lin2_W, lin2_b):
    raise NotImplementedError("write your pallas kernel here")


# speedup vs baseline: 8.3522x; 8.3522x over previous
import functools

import jax
import jax.numpy as jnp
from jax import lax
from jax.experimental import pallas as pl
from jax.experimental.pallas import tpu as pltpu
from jax.experimental.pallas import tpu_sc as plsc

NC = 2
NS = 16
NW = NC * NS
L = 16
NUM_TIMESTEPS = 2

_SC_PARAMS = pltpu.CompilerParams(use_tc_tiling_on_sc=False,
                                  needs_layout_passes=False)


def _pick_block(n, cap, mult=8):
    best = 0
    for d in range(mult, cap + 1, mult):
        if n % d == 0:
            best = d
    return best


def _leaky(v, slope=0.01):
    return jnp.maximum(v, slope * v)


def _elu(v):
    return jnp.where(v > 0, v, jnp.exp(jnp.minimum(v, 0.0)) - 1.0)


def _gru(inp, hid, WiT, WhT, bi, bh):
    H = hid.shape[-1]
    gi = jnp.dot(inp, WiT, preferred_element_type=jnp.float32) + bi
    gh = jnp.dot(hid, WhT, preferred_element_type=jnp.float32) + bh
    r = jax.nn.sigmoid(gi[:, :H] + gh[:, :H])
    z = jax.nn.sigmoid(gi[:, H:2 * H] + gh[:, H:2 * H])
    n = jnp.tanh(gi[:, 2 * H:] + r * gh[:, 2 * H:])
    return (1.0 - z) * n + z * hid


def _stage_a(N, IN, H):
    RB = _pick_block(N, 2048)

    def body(x_ref, w1_ref, b1_ref, w1a_ref, attr_ref, x1_ref, xa_ref, rd_ref):
        x1 = _leaky(jnp.dot(x_ref[...], w1_ref[...],
                            preferred_element_type=jnp.float32) + b1_ref[...])
        x1_ref[...] = x1
        xa_ref[...] = jnp.dot(x1, w1a_ref[...], preferred_element_type=jnp.float32)
        rd_ref[...] = jnp.dot(x1, attr_ref[...], preferred_element_type=jnp.float32)

    return pl.pallas_call(
        body,
        grid=(N // RB,),
        in_specs=[
            pl.BlockSpec((RB, IN), lambda i: (i, 0)),
            pl.BlockSpec((IN, H), lambda i: (0, 0)),
            pl.BlockSpec((1, H), lambda i: (0, 0)),
            pl.BlockSpec((H, H), lambda i: (0, 0)),
            pl.BlockSpec((H, 1), lambda i: (0, 0)),
        ],
        out_specs=[
            pl.BlockSpec((RB, H), lambda i: (i, 0)),
            pl.BlockSpec((RB, H), lambda i: (i, 0)),
            pl.BlockSpec((RB, 1), lambda i: (i, 0)),
        ],
        out_shape=[
            jax.ShapeDtypeStruct((N, H), jnp.float32),
            jax.ShapeDtypeStruct((N, H), jnp.float32),
            jax.ShapeDtypeStruct((N, 1), jnp.float32),
        ],
    )


def _stage_t(E, ED, H):
    EB = _pick_block(E, 4096)

    def body(ea_ref, w_ref, t_ref):
        t_ref[...] = jnp.dot(ea_ref[...], w_ref[...],
                             preferred_element_type=jnp.float32)

    return pl.pallas_call(
        body,
        grid=(E // EB,),
        in_specs=[
            pl.BlockSpec((EB, ED), lambda i: (i, 0)),
            pl.BlockSpec((ED, H), lambda i: (0, 0)),
        ],
        out_specs=pl.BlockSpec((EB, H), lambda i: (i, 0)),
        out_shape=jax.ShapeDtypeStruct((E, H), jnp.float32),
    )


def _edge_gate(N, E, H):
    EPT = E // NW
    BLK = _pick_block(EPT, 128, mult=16)
    NBLK = EPT // BLK
    ZB = _pick_block(N, BLK)
    NCH = N // ZB
    KMAX = -(-NCH // NS)
    HC = H // L
    mesh = plsc.VectorSubcoreMesh(core_axis_name="c", subcore_axis_name="s",
                                  num_cores=NC, num_subcores=NS)

    @functools.partial(
        pl.kernel,
        out_type=(jax.ShapeDtypeStruct((NC, N, H), jnp.float32),
                  jax.ShapeDtypeStruct((NC, N), jnp.float32)),
        mesh=mesh,
        compiler_params=_SC_PARAMS,
        scratch_types=[
            pltpu.VMEM((N,), jnp.float32),
            pltpu.VMEM((H,), jnp.float32),
            pltpu.VMEM((BLK,), jnp.int32),
            pltpu.VMEM((BLK,), jnp.int32),
            pltpu.VMEM((BLK, H), jnp.float32),
            pltpu.VMEM((BLK, H), jnp.float32),
            pltpu.VMEM((BLK, H), jnp.float32),
            pltpu.VMEM((BLK,), jnp.float32),
            pltpu.VMEM_SHARED((N, H), jnp.float32),
            pltpu.VMEM_SHARED((N,), jnp.float32),
            pltpu.SemaphoreType.DMA,
        ],
    )
    def kern(xa_hbm, t_hbm, rdot_hbm, attl_hbm, src_hbm, dst_hbm,
             p_hbm, z_hbm,
             rdot_v, attl_v, src_v, dst_v, rows_v, t_v, w_v, ev_v,
             p_sh, z_sh, sem):
        c = lax.axis_index("c")
        s = lax.axis_index("s")
        wid = s * NC + c

        def zrow(i, _):
            for j in range(HC):
                w_v[i, pl.ds(j * L, L)] = jnp.zeros((L,), jnp.float32)
            return _
        lax.fori_loop(0, BLK, zrow, None)

        def zev(i, _):
            ev_v[pl.ds(i * L, L)] = jnp.zeros((L,), jnp.float32)
            return _
        lax.fori_loop(0, BLK // L, zev, None)

        for k in range(KMAX):
            q = s + k * NS

            @pl.when(q < NCH)
            def _zc(q=q):
                pltpu.sync_copy(w_v.at[pl.ds(0, ZB)],
                                p_sh.at[pl.ds(q * ZB, ZB)])
                pltpu.sync_copy(ev_v.at[pl.ds(0, ZB)],
                                z_sh.at[pl.ds(q * ZB, ZB)])
        pltpu.sync_copy(rdot_hbm, rdot_v)
        pltpu.sync_copy(attl_hbm, attl_v)
        plsc.subcore_barrier()

        lane0 = lax.iota(jnp.int32, L) == 0
        zero16 = jnp.zeros((L,), jnp.float32)
        ebase = wid * EPT

        def blk_body(b, _):
            bb = ebase + b * BLK
            pltpu.sync_copy(src_hbm.at[pl.ds(bb, BLK)], src_v)
            pltpu.sync_copy(dst_hbm.at[pl.ds(bb, BLK)], dst_v)
            pltpu.async_copy(xa_hbm.at[src_v], rows_v, sem).wait()
            pltpu.sync_copy(t_hbm.at[pl.ds(bb, BLK)], t_v)

            def grp(g, _):
                d16 = dst_v[pl.ds(g * L, L)]
                rd16 = plsc.load_gather(rdot_v, [d16])
                for i in range(L):
                    ei = g * L + i
                    acc = zero16
                    ms = []
                    for j in range(HC):
                        gg = rows_v[ei, pl.ds(j * L, L)] + t_v[ei, pl.ds(j * L, L)]
                        m = jnp.maximum(gg, 0.01 * gg)
                        ms.append(m)
                        acc = acc + m * attl_v[pl.ds(j * L, L)]
                    sc = jnp.sum(acc) + rd16[i]
                    aa = jnp.maximum(sc, 0.01 * sc)
                    ev = jnp.exp(jnp.broadcast_to(aa, (L,)))
                    for j in range(HC):
                        w_v[ei, pl.ds(j * L, L)] = ms[j] * ev
                    plsc.store_scatter(ev_v, [jnp.broadcast_to(ei, (L,))],
                                       ev, mask=lane0)
                return _
            lax.fori_loop(0, BLK // L, grp, None)
            pltpu.sync_copy(w_v, p_sh.at[dst_v], add=True)
            pltpu.sync_copy(ev_v, z_sh.at[dst_v], add=True)
            return _
        lax.fori_loop(0, NBLK, blk_body, None)
        plsc.subcore_barrier()
        for k in range(KMAX):
            q = s + k * NS

            @pl.when(q < NCH)
            def _oc(q=q):
                pltpu.sync_copy(p_sh.at[pl.ds(q * ZB, ZB)],
                                p_hbm.at[c, pl.ds(q * ZB, ZB)])
                pltpu.sync_copy(z_sh.at[pl.ds(q * ZB, ZB)],
                                z_hbm.at[c, pl.ds(q * ZB, ZB)])

    return kern


def _edge_gat(N, E, H):
    EPT = E // NW
    BLK = _pick_block(EPT, 128, mult=16)
    NBLK = EPT // BLK
    ZB = _pick_block(N, BLK)
    NCH = N // ZB
    KMAX = -(-NCH // NS)
    HC = H // L
    mesh = plsc.VectorSubcoreMesh(core_axis_name="c", subcore_axis_name="s",
                                  num_cores=NC, num_subcores=NS)

    @functools.partial(
        pl.kernel,
        out_type=(jax.ShapeDtypeStruct((NC, N, H), jnp.float32),
                  jax.ShapeDtypeStruct((NC, N), jnp.float32)),
        mesh=mesh,
        compiler_params=_SC_PARAMS,
        scratch_types=[
            pltpu.VMEM((N,), jnp.float32),
            pltpu.VMEM((N,), jnp.float32),
            pltpu.VMEM((BLK,), jnp.int32),
            pltpu.VMEM((BLK,), jnp.int32),
            pltpu.VMEM((BLK, H), jnp.float32),
            pltpu.VMEM((BLK, H), jnp.float32),
            pltpu.VMEM((BLK,), jnp.float32),
            pltpu.VMEM_SHARED((N, H), jnp.float32),
            pltpu.VMEM_SHARED((N,), jnp.float32),
            pltpu.SemaphoreType.DMA,
        ],
    )
    def kern(xp_hbm, asrc_hbm, adst_hbm, src_hbm, dst_hbm,
             p_hbm, z_hbm,
             asrc_v, adst_v, src_v, dst_v, rows_v, w_v, ev_v,
             p_sh, z_sh, sem):
        c = lax.axis_index("c")
        s = lax.axis_index("s")
        wid = s * NC + c

        def zrow(i, _):
            for j in range(HC):
                w_v[i, pl.ds(j * L, L)] = jnp.zeros((L,), jnp.float32)
            return _
        lax.fori_loop(0, BLK, zrow, None)

        def zev(i, _):
            ev_v[pl.ds(i * L, L)] = jnp.zeros((L,), jnp.float32)
            return _
        lax.fori_loop(0, BLK // L, zev, None)

        for k in range(KMAX):
            q = s + k * NS

            @pl.when(q < NCH)
            def _zc(q=q):
                pltpu.sync_copy(w_v.at[pl.ds(0, ZB)],
                                p_sh.at[pl.ds(q * ZB, ZB)])
                pltpu.sync_copy(ev_v.at[pl.ds(0, ZB)],
                                z_sh.at[pl.ds(q * ZB, ZB)])
        pltpu.sync_copy(asrc_hbm, asrc_v)
        pltpu.sync_copy(adst_hbm, adst_v)
        plsc.subcore_barrier()

        ebase = wid * EPT

        def blk_body(b, _):
            bb = ebase + b * BLK
            pltpu.sync_copy(src_hbm.at[pl.ds(bb, BLK)], src_v)
            pltpu.sync_copy(dst_hbm.at[pl.ds(bb, BLK)], dst_v)
            pltpu.async_copy(xp_hbm.at[src_v], rows_v, sem).wait()

            def grp(g, _):
                s16 = src_v[pl.ds(g * L, L)]
                d16 = dst_v[pl.ds(g * L, L)]
                a0 = (plsc.load_gather(asrc_v, [s16])
                      + plsc.load_gather(adst_v, [d16]))
                aa = jnp.maximum(a0, 0.01 * a0)
                ev16 = jnp.exp(aa)
                ev_v[pl.ds(g * L, L)] = ev16
                for i in range(L):
                    ei = g * L + i
                    ev = jnp.broadcast_to(ev16[i], (L,))
                    for j in range(HC):
                        w_v[ei, pl.ds(j * L, L)] = rows_v[ei, pl.ds(j * L, L)] * ev
                return _
            lax.fori_loop(0, BLK // L, grp, None)
            pltpu.sync_copy(w_v, p_sh.at[dst_v], add=True)
            pltpu.sync_copy(ev_v, z_sh.at[dst_v], add=True)
            return _
        lax.fori_loop(0, NBLK, blk_body, None)
        plsc.subcore_barrier()
        for k in range(KMAX):
            q = s + k * NS

            @pl.when(q < NCH)
            def _oc(q=q):
                pltpu.sync_copy(p_sh.at[pl.ds(q * ZB, ZB)],
                                p_hbm.at[c, pl.ds(q * ZB, ZB)])
                pltpu.sync_copy(z_sh.at[pl.ds(q * ZB, ZB)],
                                z_hbm.at[c, pl.ds(q * ZB, ZB)])

    return kern


def _stage_c(N, H):
    RB = _pick_block(N, 2048)

    def body(p1_ref, z1_ref, x1_ref, w2_ref, gb_ref, wi_ref, wh_ref, bi_ref,
             bh_ref, gw_ref, as_ref, ad_ref, x2_ref, xp_ref, asrc_ref, adst_ref):
        S = p1_ref[0] + p1_ref[1]
        Z = z1_ref[0] + z1_ref[1]
        agg = jnp.dot(S / (Z + 1e-16), w2_ref[...],
                      preferred_element_type=jnp.float32) + gb_ref[...]
        h = _elu(agg)
        x1 = x1_ref[...]
        x2 = jnp.maximum(_gru(h, x1, wi_ref[...], wh_ref[...],
                              bi_ref[...], bh_ref[...]), 0.0)
        xp = jnp.dot(x2, gw_ref[...], preferred_element_type=jnp.float32)
        x2_ref[...] = x2
        xp_ref[...] = xp
        asrc_ref[...] = jnp.dot(xp, as_ref[...], preferred_element_type=jnp.float32)
        adst_ref[...] = jnp.dot(xp, ad_ref[...], preferred_element_type=jnp.float32)

    return pl.pallas_call(
        body,
        grid=(N // RB,),
        in_specs=[
            pl.BlockSpec((NC, RB, H), lambda i: (0, i, 0)),
            pl.BlockSpec((NC, RB, 1), lambda i: (0, i, 0)),
            pl.BlockSpec((RB, H), lambda i: (i, 0)),
            pl.BlockSpec((H, H), lambda i: (0, 0)),
            pl.BlockSpec((1, H), lambda i: (0, 0)),
            pl.BlockSpec((H, 3 * H), lambda i: (0, 0)),
            pl.BlockSpec((H, 3 * H), lambda i: (0, 0)),
            pl.BlockSpec((1, 3 * H), lambda i: (0, 0)),
            pl.BlockSpec((1, 3 * H), lambda i: (0, 0)),
            pl.BlockSpec((H, H), lambda i: (0, 0)),
            pl.BlockSpec((H, 1), lambda i: (0, 0)),
            pl.BlockSpec((H, 1), lambda i: (0, 0)),
        ],
        out_specs=[
            pl.BlockSpec((RB, H), lambda i: (i, 0)),
            pl.BlockSpec((RB, H), lambda i: (i, 0)),
            pl.BlockSpec((RB, 1), lambda i: (i, 0)),
            pl.BlockSpec((RB, 1), lambda i: (i, 0)),
        ],
        out_shape=[
            jax.ShapeDtypeStruct((N, H), jnp.float32),
            jax.ShapeDtypeStruct((N, H), jnp.float32),
            jax.ShapeDtypeStruct((N, 1), jnp.float32),
            jax.ShapeDtypeStruct((N, 1), jnp.float32),
        ],
    )


def _stage_e1(N, H):
    RB = _pick_block(N, 2048)

    def body(p2_ref, z2_ref, x2_ref, gb_ref, wi_ref, wh_ref, bi_ref, bh_ref,
             mw_ref, ms_ref, x3_ref, xs_ref, amol_ref):
        S = p2_ref[0] + p2_ref[1]
        Z = z2_ref[0] + z2_ref[1]
        h = _elu(S / (Z + 1e-16) + gb_ref[...])
        x2 = x2_ref[...]
        x3 = jnp.maximum(_gru(h, x2, wi_ref[...], wh_ref[...],
                              bi_ref[...], bh_ref[...]), 0.0)
        xs = jnp.dot(x3, mw_ref[...], preferred_element_type=jnp.float32)
        x3_ref[...] = x3
        xs_ref[...] = xs
        amol_ref[...] = jnp.dot(xs, ms_ref[...], preferred_element_type=jnp.float32)

    return pl.pallas_call(
        body,
        grid=(N // RB,),
        in_specs=[
            pl.BlockSpec((NC, RB, H), lambda i: (0, i, 0)),
            pl.BlockSpec((NC, RB, 1), lambda i: (0, i, 0)),
            pl.BlockSpec((RB, H), lambda i: (i, 0)),
            pl.BlockSpec((1, H), lambda i: (0, 0)),
            pl.BlockSpec((H, 3 * H), lambda i: (0, 0)),
            pl.BlockSpec((H, 3 * H), lambda i: (0, 0)),
            pl.BlockSpec((1, 3 * H), lambda i: (0, 0)),
            pl.BlockSpec((1, 3 * H), lambda i: (0, 0)),
            pl.BlockSpec((H, H), lambda i: (0, 0)),
            pl.BlockSpec((H, 1), lambda i: (0, 0)),
        ],
        out_specs=[
            pl.BlockSpec((RB, H), lambda i: (i, 0)),
            pl.BlockSpec((RB, H), lambda i: (i, 0)),
            pl.BlockSpec((RB, 1), lambda i: (i, 0)),
        ],
        out_shape=[
            jax.ShapeDtypeStruct((N, H), jnp.float32),
            jax.ShapeDtypeStruct((N, H), jnp.float32),
            jax.ShapeDtypeStruct((N, 1), jnp.float32),
        ],
    )


def _stage_e2(N, H, Bn, OUT):
    c00 = (((0,), (0,)), ((), ()))

    def body(x3_ref, xs_ref, amol_ref, bt_ref, mw_ref, md_ref, mb_ref,
             wi_ref, wh_ref, bi_ref, bh_ref, l2_ref, l2b_ref, res_ref):
        bt = bt_ref[...]
        iot = lax.broadcasted_iota(jnp.int32, (N, Bn), 1)
        Mt = (bt == iot).astype(jnp.float32)
        x3 = x3_ref[...]
        out = jnp.maximum(
            lax.dot_general(Mt, x3, c00, preferred_element_type=jnp.float32), 0.0)
        xs = xs_ref[...]
        amol = amol_ref[...]
        for _ in range(NUM_TIMESTEPS):
            od = jnp.dot(out, mw_ref[...], preferred_element_type=jnp.float32)
            adm = jnp.dot(od, md_ref[...], preferred_element_type=jnp.float32)
            a0 = amol + jnp.dot(Mt, adm, preferred_element_type=jnp.float32)
            e = jnp.exp(jnp.maximum(a0, 0.01 * a0))
            Zb = lax.dot_general(Mt, e, c00, preferred_element_type=jnp.float32)
            Pb = lax.dot_general(Mt, xs * e, c00, preferred_element_type=jnp.float32)
            h = _elu(Pb / (Zb + 1e-16) + mb_ref[...])
            out = jnp.maximum(_gru(h, out, wi_ref[...], wh_ref[...],
                                   bi_ref[...], bh_ref[...]), 0.0)
        res_ref[...] = jnp.dot(out, l2_ref[...],
                               preferred_element_type=jnp.float32) + l2b_ref[...]

    return pl.pallas_call(
        body,
        out_shape=jax.ShapeDtypeStruct((Bn, OUT), jnp.float32),
    )


def _run(x, edge_index, edge_attr, batch, Bn,
         lin1_W, lin1_b, att_l, att_r, gate_lin1_W, gate_lin2_W, gate_bias,
         gru0_Wi, gru0_Wh, gru0_bi, gru0_bh,
         gat_W, gat_att_src, gat_att_dst, gat_bias,
         gru1_Wi, gru1_Wh, gru1_bi, gru1_bh,
         mol_W, mol_att_src, mol_att_dst, mol_bias,
         mgru_Wi, mgru_Wh, mgru_bi, mgru_bh,
         lin2_W, lin2_b):
    N, IN = x.shape
    H = lin1_W.shape[0]
    E = edge_index.shape[1]
    ED = edge_attr.shape[1]
    OUT = lin2_W.shape[0]

    src = edge_index[0]
    dst = edge_index[1]
    W1a = gate_lin1_W[:, :H]
    W1b = gate_lin1_W[:, H:]

    x1, xa, rdot = _stage_a(N, IN, H)(
        x, lin1_W.T, lin1_b[None, :], W1a.T, att_r[:, None])
    t = _stage_t(E, ED, H)(edge_attr, W1b.T)
    p1, z1 = _edge_gate(N, E, H)(xa, t, rdot.reshape(-1), att_l, src, dst)
    x2, xp, asrc, adst = _stage_c(N, H)(
        p1, z1[:, :, None], x1, gate_lin2_W.T, gate_bias[None, :],
        gru0_Wi.T, gru0_Wh.T, gru0_bi[None, :], gru0_bh[None, :],
        gat_W.T, gat_att_src[:, None], gat_att_dst[:, None])
    p2, z2 = _edge_gat(N, E, H)(xp, asrc.reshape(-1), adst.reshape(-1), src, dst)
    x3, xs, amol = _stage_e1(N, H)(
        p2, z2[:, :, None], x2, gat_bias[None, :],
        gru1_Wi.T, gru1_Wh.T, gru1_bi[None, :], gru1_bh[None, :],
        mol_W.T, mol_att_src[:, None])
    res = _stage_e2(N, H, Bn, OUT)(
        x3, xs, amol, batch[:, None].astype(jnp.int32),
        mol_W.T, mol_att_dst[:, None], mol_bias[None, :],
        mgru_Wi.T, mgru_Wh.T, mgru_bi[None, :], mgru_bh[None, :],
        lin2_W.T, lin2_b[None, :])
    return res


def kernel(x, edge_index, edge_attr, batch,
           lin1_W, lin1_b, att_l, att_r, gate_lin1_W, gate_lin2_W, gate_bias,
           gru0_Wi, gru0_Wh, gru0_bi, gru0_bh,
           gat_W, gat_att_src, gat_att_dst, gat_bias,
           gru1_Wi, gru1_Wh, gru1_bi, gru1_bh,
           mol_W, mol_att_src, mol_att_dst, mol_bias,
           mgru_Wi, mgru_Wh, mgru_bi, mgru_bh,
           lin2_W, lin2_b):
    return _run(x, edge_index, edge_attr, batch, 64,
                lin1_W, lin1_b, att_l, att_r, gate_lin1_W, gate_lin2_W,
                gate_bias, gru0_Wi, gru0_Wh, gru0_bi, gru0_bh,
                gat_W, gat_att_src, gat_att_dst, gat_bias,
                gru1_Wi, gru1_Wh, gru1_bi, gru1_bh,
                mol_W, mol_att_src, mol_att_dst, mol_bias,
                mgru_Wi, mgru_Wh, mgru_bi, mgru_bh,
                lin2_W, lin2_b)


# speedup vs baseline: 10.8594x; 1.3002x over previous
import functools

import jax
import jax.numpy as jnp
from jax import lax
from jax.experimental import pallas as pl
from jax.experimental.pallas import tpu as pltpu
from jax.experimental.pallas import tpu_sc as plsc

NC = 2
NS = 16
NW = NC * NS
L = 16
NUM_TIMESTEPS = 2

_SC_PARAMS = pltpu.CompilerParams(use_tc_tiling_on_sc=False,
                                  needs_layout_passes=False)


def _pick_block(n, cap, mult=8):
    best = 0
    for d in range(mult, cap + 1, mult):
        if n % d == 0:
            best = d
    return best


def _leaky(v, slope=0.01):
    return jnp.maximum(v, slope * v)


def _elu(v):
    return jnp.where(v > 0, v, jnp.exp(jnp.minimum(v, 0.0)) - 1.0)


def _gru(inp, hid, WiT, WhT, bi, bh):
    H = hid.shape[-1]
    gi = jnp.dot(inp, WiT, preferred_element_type=jnp.float32,
                 precision=lax.Precision.HIGHEST) + bi
    gh = jnp.dot(hid, WhT, preferred_element_type=jnp.float32,
                 precision=lax.Precision.HIGHEST) + bh
    r = jax.nn.sigmoid(gi[:, :H] + gh[:, :H])
    z = jax.nn.sigmoid(gi[:, H:2 * H] + gh[:, H:2 * H])
    n = jnp.tanh(gi[:, 2 * H:] + r * gh[:, 2 * H:])
    return (1.0 - z) * n + z * hid


def _stage_a(N, IN, H):
    RB = _pick_block(N, 2048)

    def body(x_ref, w1_ref, b1_ref, w1a_ref, attr_ref, x1_ref, xa_ref, rd_ref):
        x1 = _leaky(jnp.dot(x_ref[...], w1_ref[...],
                            preferred_element_type=jnp.float32,
                 precision=lax.Precision.HIGHEST) + b1_ref[...])
        x1_ref[...] = x1
        xa_ref[...] = jnp.dot(x1, w1a_ref[...], preferred_element_type=jnp.float32,
                 precision=lax.Precision.HIGHEST)
        rd_ref[...] = jnp.dot(x1, attr_ref[...], preferred_element_type=jnp.float32,
                 precision=lax.Precision.HIGHEST)

    return pl.pallas_call(
        body,
        grid=(N // RB,),
        in_specs=[
            pl.BlockSpec((RB, IN), lambda i: (i, 0)),
            pl.BlockSpec((IN, H), lambda i: (0, 0)),
            pl.BlockSpec((1, H), lambda i: (0, 0)),
            pl.BlockSpec((H, H), lambda i: (0, 0)),
            pl.BlockSpec((H, 1), lambda i: (0, 0)),
        ],
        out_specs=[
            pl.BlockSpec((RB, H), lambda i: (i, 0)),
            pl.BlockSpec((RB, H), lambda i: (i, 0)),
            pl.BlockSpec((RB, 1), lambda i: (i, 0)),
        ],
        out_shape=[
            jax.ShapeDtypeStruct((N, H), jnp.float32),
            jax.ShapeDtypeStruct((N, H), jnp.float32),
            jax.ShapeDtypeStruct((N, 1), jnp.float32),
        ],
    )


def _stage_t(E, ED, H):
    EB = _pick_block(E, 4096)

    def body(ea_ref, w_ref, t_ref):
        t_ref[...] = jnp.dot(ea_ref[...], w_ref[...],
                             preferred_element_type=jnp.float32,
                 precision=lax.Precision.HIGHEST)

    return pl.pallas_call(
        body,
        grid=(E // EB,),
        in_specs=[
            pl.BlockSpec((EB, ED), lambda i: (i, 0)),
            pl.BlockSpec((ED, H), lambda i: (0, 0)),
        ],
        out_specs=pl.BlockSpec((EB, H), lambda i: (i, 0)),
        out_shape=jax.ShapeDtypeStruct((E, H), jnp.float32),
    )


def _edge_gate(N, E, H):
    EPT = E // NW
    BLK = _pick_block(EPT, 128, mult=16)
    NBLK = EPT // BLK
    ZB = _pick_block(N, BLK)
    NCH = N // ZB
    KMAX = -(-NCH // NS)
    HC = H // L
    mesh = plsc.VectorSubcoreMesh(core_axis_name="c", subcore_axis_name="s",
                                  num_cores=NC, num_subcores=NS)

    @functools.partial(
        pl.kernel,
        out_type=(jax.ShapeDtypeStruct((NC, N, H), jnp.float32),
                  jax.ShapeDtypeStruct((NC, N), jnp.float32)),
        mesh=mesh,
        compiler_params=_SC_PARAMS,
        scratch_types=[
            pltpu.VMEM((N,), jnp.float32),
            pltpu.VMEM((H,), jnp.float32),
            pltpu.VMEM((BLK,), jnp.int32),
            pltpu.VMEM((BLK,), jnp.int32),
            pltpu.VMEM((BLK,), jnp.int32),
            pltpu.VMEM((BLK,), jnp.int32),
            pltpu.VMEM((BLK,), jnp.int32),
            pltpu.VMEM((BLK,), jnp.int32),
            pltpu.VMEM((2, BLK, H), jnp.float32),
            pltpu.VMEM((BLK, H), jnp.float32),
            pltpu.VMEM((2, BLK), jnp.float32),
            pltpu.VMEM_SHARED((N, H), jnp.float32),
            pltpu.VMEM_SHARED((N,), jnp.float32),
            pltpu.SemaphoreType.DMA((2,)),
            pltpu.SemaphoreType.DMA((2,)),
            pltpu.SemaphoreType.DMA,
        ],
    )
    def kern(xa_hbm, t_hbm, rdot_hbm, attl_hbm, src_hbm, dst_hbm,
             p_hbm, z_hbm,
             rdot_v, attl_v, src0_v, src1_v, dst0_v, dst1_v,
             sdst0_v, sdst1_v, rows_v, t_v, ev_v,
             p_sh, z_sh, sem_idx, sem_rows, sem_t):
        c = lax.axis_index("c")
        s = lax.axis_index("s")
        wid = s * NC + c
        srcs = [src0_v, src1_v]
        dsts = [dst0_v, dst1_v]
        sdsts = [sdst0_v, sdst1_v]

        def zrow(i, _):
            for j in range(HC):
                rows_v[0, i, pl.ds(j * L, L)] = jnp.zeros((L,), jnp.float32)
            return _
        lax.fori_loop(0, BLK, zrow, None)

        def zev(i, _):
            ev_v[0, pl.ds(i * L, L)] = jnp.zeros((L,), jnp.float32)
            return _
        lax.fori_loop(0, BLK // L, zev, None)

        for k in range(KMAX):
            q = s + k * NS

            @pl.when(q < NCH)
            def _zc(q=q):
                pltpu.sync_copy(rows_v.at[0, pl.ds(0, ZB)],
                                p_sh.at[pl.ds(q * ZB, ZB)])
                pltpu.sync_copy(ev_v.at[0, pl.ds(0, ZB)],
                                z_sh.at[pl.ds(q * ZB, ZB)])

        pltpu.sync_copy(rdot_hbm, rdot_v)
        pltpu.sync_copy(attl_hbm, attl_v)
        plsc.subcore_barrier()

        lane0 = lax.iota(jnp.int32, L) == 0
        zero16 = jnp.zeros((L,), jnp.float32)
        ebase = wid * EPT

        def issue_idx(b, buf):
            off = ebase + b * BLK
            pltpu.async_copy(src_hbm.at[pl.ds(off, BLK)], srcs[buf],
                             sem_idx.at[buf])
            pltpu.async_copy(dst_hbm.at[pl.ds(off, BLK)], dsts[buf],
                             sem_idx.at[buf])

        def wait_idx(buf):
            pltpu.make_async_copy(src_hbm.at[pl.ds(0, BLK)], srcs[buf],
                                  sem_idx.at[buf]).wait()
            pltpu.make_async_copy(dst_hbm.at[pl.ds(0, BLK)], dsts[buf],
                                  sem_idx.at[buf]).wait()

        def issue_fetch(b, buf):
            pltpu.async_copy(xa_hbm.at[srcs[buf]], rows_v.at[buf],
                             sem_rows.at[buf])

        def wait_fetch(buf):
            pltpu.make_async_copy(xa_hbm.at[srcs[buf]], rows_v.at[buf],
                                  sem_rows.at[buf]).wait()

        def issue_t(b):
            pltpu.async_copy(t_hbm.at[pl.ds(ebase + b * BLK, BLK)],
                             t_v, sem_t)

        def wait_t():
            pltpu.make_async_copy(t_hbm.at[pl.ds(0, BLK)], t_v,
                                  sem_t).wait()

        def emit_iter(b, buf):
            nbuf = 1 - buf
            bq = jnp.int32(b)
            wait_fetch(buf)
            wait_t()
            for i in range(BLK // L):
                sdsts[buf][pl.ds(i * L, L)] = dsts[buf][pl.ds(i * L, L)]

            @pl.when(bq + 2 < NBLK)
            def _pi():
                issue_idx(b + 2, buf)

            @pl.when(bq + 1 < NBLK)
            def _pf():
                wait_idx(nbuf)
                issue_fetch(b + 1, nbuf)

            lanes = lax.iota(jnp.int32, L)

            def grp(g, _):
                d16 = sdsts[buf][pl.ds(g * L, L)]
                rd16 = plsc.load_gather(rdot_v, [d16])
                ev_acc = zero16
                for i in range(L):
                    ei = g * L + i
                    acc = zero16
                    ms = []
                    for j in range(HC):
                        gg = (rows_v[buf, ei, pl.ds(j * L, L)]
                              + t_v[ei, pl.ds(j * L, L)])
                        m = jnp.maximum(gg, 0.01 * gg)
                        ms.append(m)
                        acc = acc + m * attl_v[pl.ds(j * L, L)]
                    sc = jnp.sum(acc) + rd16[i]
                    aa = jnp.maximum(sc, 0.01 * sc)
                    ev = jnp.exp(jnp.broadcast_to(aa, (L,)))
                    for j in range(HC):
                        rows_v[buf, ei, pl.ds(j * L, L)] = ms[j] * ev
                    ev_acc = jnp.where(lanes == i, ev, ev_acc)
                ev_v[buf, pl.ds(g * L, L)] = ev_acc
                return _
            lax.fori_loop(0, BLK // L, grp, None)

            @pl.when(bq + 1 < NBLK)
            def _pt():
                issue_t(b + 1)
            pltpu.sync_copy(rows_v.at[buf], p_sh.at[sdsts[buf]], add=True)
            pltpu.sync_copy(ev_v.at[buf], z_sh.at[sdsts[buf]], add=True)

        issue_idx(0, 0)
        if NBLK > 1:
            issue_idx(1, 1)
        wait_idx(0)
        issue_fetch(0, 0)
        issue_t(0)

        def pair_body(k, _):
            emit_iter(2 * k, 0)
            emit_iter(2 * k + 1, 1)
            return _
        lax.fori_loop(0, NBLK // 2, pair_body, None)
        for b in range((NBLK // 2) * 2, NBLK):
            emit_iter(b, b % 2)

        plsc.subcore_barrier()
        for k in range(KMAX):
            q = s + k * NS

            @pl.when(q < NCH)
            def _oc(q=q):
                pltpu.sync_copy(p_sh.at[pl.ds(q * ZB, ZB)],
                                p_hbm.at[c, pl.ds(q * ZB, ZB)])
                pltpu.sync_copy(z_sh.at[pl.ds(q * ZB, ZB)],
                                z_hbm.at[c, pl.ds(q * ZB, ZB)])

    return kern


def _edge_gat(N, E, H):
    EPT = E // NW
    BLK = _pick_block(EPT, 128, mult=16)
    NBLK = EPT // BLK
    ZB = _pick_block(N, BLK)
    NCH = N // ZB
    KMAX = -(-NCH // NS)
    HC = H // L
    mesh = plsc.VectorSubcoreMesh(core_axis_name="c", subcore_axis_name="s",
                                  num_cores=NC, num_subcores=NS)

    @functools.partial(
        pl.kernel,
        out_type=(jax.ShapeDtypeStruct((NC, N, H), jnp.float32),
                  jax.ShapeDtypeStruct((NC, N), jnp.float32)),
        mesh=mesh,
        compiler_params=_SC_PARAMS,
        scratch_types=[
            pltpu.VMEM((N,), jnp.float32),
            pltpu.VMEM((N,), jnp.float32),
            pltpu.VMEM((BLK,), jnp.int32),
            pltpu.VMEM((BLK,), jnp.int32),
            pltpu.VMEM((BLK,), jnp.int32),
            pltpu.VMEM((BLK,), jnp.int32),
            pltpu.VMEM((BLK,), jnp.int32),
            pltpu.VMEM((BLK,), jnp.int32),
            pltpu.VMEM((2, BLK, H), jnp.float32),
            pltpu.VMEM((2, BLK), jnp.float32),
            pltpu.VMEM_SHARED((N, H), jnp.float32),
            pltpu.VMEM_SHARED((N,), jnp.float32),
            pltpu.SemaphoreType.DMA((2,)),
            pltpu.SemaphoreType.DMA((2,)),
        ],
    )
    def kern(xp_hbm, asrc_hbm, adst_hbm, src_hbm, dst_hbm,
             p_hbm, z_hbm,
             asrc_v, adst_v, src0_v, src1_v, dst0_v, dst1_v,
             sdst0_v, sdst1_v, rows_v, ev_v,
             p_sh, z_sh, sem_idx, sem_rows):
        c = lax.axis_index("c")
        s = lax.axis_index("s")
        wid = s * NC + c
        srcs = [src0_v, src1_v]
        dsts = [dst0_v, dst1_v]
        sdsts = [sdst0_v, sdst1_v]

        def zrow(i, _):
            for j in range(HC):
                rows_v[0, i, pl.ds(j * L, L)] = jnp.zeros((L,), jnp.float32)
            return _
        lax.fori_loop(0, BLK, zrow, None)

        def zev(i, _):
            ev_v[0, pl.ds(i * L, L)] = jnp.zeros((L,), jnp.float32)
            return _
        lax.fori_loop(0, BLK // L, zev, None)

        for k in range(KMAX):
            q = s + k * NS

            @pl.when(q < NCH)
            def _zc(q=q):
                pltpu.sync_copy(rows_v.at[0, pl.ds(0, ZB)],
                                p_sh.at[pl.ds(q * ZB, ZB)])
                pltpu.sync_copy(ev_v.at[0, pl.ds(0, ZB)],
                                z_sh.at[pl.ds(q * ZB, ZB)])

        pltpu.sync_copy(asrc_hbm, asrc_v)
        pltpu.sync_copy(adst_hbm, adst_v)
        plsc.subcore_barrier()

        ebase = wid * EPT

        def issue_idx(b, buf):
            off = ebase + b * BLK
            pltpu.async_copy(src_hbm.at[pl.ds(off, BLK)], srcs[buf],
                             sem_idx.at[buf])
            pltpu.async_copy(dst_hbm.at[pl.ds(off, BLK)], dsts[buf],
                             sem_idx.at[buf])

        def wait_idx(buf):
            pltpu.make_async_copy(src_hbm.at[pl.ds(0, BLK)], srcs[buf],
                                  sem_idx.at[buf]).wait()
            pltpu.make_async_copy(dst_hbm.at[pl.ds(0, BLK)], dsts[buf],
                                  sem_idx.at[buf]).wait()

        def issue_fetch(b, buf):
            pltpu.async_copy(xp_hbm.at[srcs[buf]], rows_v.at[buf],
                             sem_rows.at[buf])

        def wait_fetch(buf):
            pltpu.make_async_copy(xp_hbm.at[srcs[buf]], rows_v.at[buf],
                                  sem_rows.at[buf]).wait()

        def emit_iter(b, buf):
            nbuf = 1 - buf
            bq = jnp.int32(b)
            wait_fetch(buf)

            def agrp(g, _):
                s16 = srcs[buf][pl.ds(g * L, L)]
                d16 = dsts[buf][pl.ds(g * L, L)]
                a0 = (plsc.load_gather(asrc_v, [s16])
                      + plsc.load_gather(adst_v, [d16]))
                aa = jnp.maximum(a0, 0.01 * a0)
                ev_v[buf, pl.ds(g * L, L)] = jnp.exp(aa)
                sdsts[buf][pl.ds(g * L, L)] = d16
                return _
            lax.fori_loop(0, BLK // L, agrp, None)

            @pl.when(bq + 2 < NBLK)
            def _pi():
                issue_idx(b + 2, buf)

            @pl.when(bq + 1 < NBLK)
            def _pf():
                wait_idx(nbuf)
                issue_fetch(b + 1, nbuf)

            def grp(g, _):
                ev16 = ev_v[buf, pl.ds(g * L, L)]
                for i in range(L):
                    ei = g * L + i
                    ev = jnp.broadcast_to(ev16[i], (L,))
                    for j in range(HC):
                        rows_v[buf, ei, pl.ds(j * L, L)] = (
                            rows_v[buf, ei, pl.ds(j * L, L)] * ev)
                return _
            lax.fori_loop(0, BLK // L, grp, None)
            pltpu.sync_copy(rows_v.at[buf], p_sh.at[sdsts[buf]], add=True)
            pltpu.sync_copy(ev_v.at[buf], z_sh.at[sdsts[buf]], add=True)

        issue_idx(0, 0)
        if NBLK > 1:
            issue_idx(1, 1)
        wait_idx(0)
        issue_fetch(0, 0)

        def pair_body(k, _):
            emit_iter(2 * k, 0)
            emit_iter(2 * k + 1, 1)
            return _
        lax.fori_loop(0, NBLK // 2, pair_body, None)
        for b in range((NBLK // 2) * 2, NBLK):
            emit_iter(b, b % 2)

        plsc.subcore_barrier()
        for k in range(KMAX):
            q = s + k * NS

            @pl.when(q < NCH)
            def _oc(q=q):
                pltpu.sync_copy(p_sh.at[pl.ds(q * ZB, ZB)],
                                p_hbm.at[c, pl.ds(q * ZB, ZB)])
                pltpu.sync_copy(z_sh.at[pl.ds(q * ZB, ZB)],
                                z_hbm.at[c, pl.ds(q * ZB, ZB)])

    return kern


def _stage_c(N, H):
    RB = _pick_block(N, 2048)

    def body(p1_ref, z1_ref, x1_ref, w2_ref, gb_ref, wi_ref, wh_ref, bi_ref,
             bh_ref, gw_ref, as_ref, ad_ref, x2_ref, xp_ref, asrc_ref, adst_ref):
        S = p1_ref[0] + p1_ref[1]
        Z = z1_ref[0] + z1_ref[1]
        agg = jnp.dot(S / (Z + 1e-16), w2_ref[...],
                      preferred_element_type=jnp.float32,
                 precision=lax.Precision.HIGHEST) + gb_ref[...]
        h = _elu(agg)
        x1 = x1_ref[...]
        x2 = jnp.maximum(_gru(h, x1, wi_ref[...], wh_ref[...],
                              bi_ref[...], bh_ref[...]), 0.0)
        xp = jnp.dot(x2, gw_ref[...], preferred_element_type=jnp.float32,
                 precision=lax.Precision.HIGHEST)
        x2_ref[...] = x2
        xp_ref[...] = xp
        asrc_ref[...] = jnp.dot(xp, as_ref[...], preferred_element_type=jnp.float32,
                 precision=lax.Precision.HIGHEST)
        adst_ref[...] = jnp.dot(xp, ad_ref[...], preferred_element_type=jnp.float32,
                 precision=lax.Precision.HIGHEST)

    return pl.pallas_call(
        body,
        grid=(N // RB,),
        in_specs=[
            pl.BlockSpec((NC, RB, H), lambda i: (0, i, 0)),
            pl.BlockSpec((NC, RB, 1), lambda i: (0, i, 0)),
            pl.BlockSpec((RB, H), lambda i: (i, 0)),
            pl.BlockSpec((H, H), lambda i: (0, 0)),
            pl.BlockSpec((1, H), lambda i: (0, 0)),
            pl.BlockSpec((H, 3 * H), lambda i: (0, 0)),
            pl.BlockSpec((H, 3 * H), lambda i: (0, 0)),
            pl.BlockSpec((1, 3 * H), lambda i: (0, 0)),
            pl.BlockSpec((1, 3 * H), lambda i: (0, 0)),
            pl.BlockSpec((H, H), lambda i: (0, 0)),
            pl.BlockSpec((H, 1), lambda i: (0, 0)),
            pl.BlockSpec((H, 1), lambda i: (0, 0)),
        ],
        out_specs=[
            pl.BlockSpec((RB, H), lambda i: (i, 0)),
            pl.BlockSpec((RB, H), lambda i: (i, 0)),
            pl.BlockSpec((RB, 1), lambda i: (i, 0)),
            pl.BlockSpec((RB, 1), lambda i: (i, 0)),
        ],
        out_shape=[
            jax.ShapeDtypeStruct((N, H), jnp.float32),
            jax.ShapeDtypeStruct((N, H), jnp.float32),
            jax.ShapeDtypeStruct((N, 1), jnp.float32),
            jax.ShapeDtypeStruct((N, 1), jnp.float32),
        ],
    )


def _stage_e1(N, H):
    RB = _pick_block(N, 2048)

    def body(p2_ref, z2_ref, x2_ref, gb_ref, wi_ref, wh_ref, bi_ref, bh_ref,
             mw_ref, ms_ref, x3_ref, xs_ref, amol_ref):
        S = p2_ref[0] + p2_ref[1]
        Z = z2_ref[0] + z2_ref[1]
        h = _elu(S / (Z + 1e-16) + gb_ref[...])
        x2 = x2_ref[...]
        x3 = jnp.maximum(_gru(h, x2, wi_ref[...], wh_ref[...],
                              bi_ref[...], bh_ref[...]), 0.0)
        xs = jnp.dot(x3, mw_ref[...], preferred_element_type=jnp.float32,
                 precision=lax.Precision.HIGHEST)
        x3_ref[...] = x3
        xs_ref[...] = xs
        amol_ref[...] = jnp.dot(xs, ms_ref[...], preferred_element_type=jnp.float32,
                 precision=lax.Precision.HIGHEST)

    return pl.pallas_call(
        body,
        grid=(N // RB,),
        in_specs=[
            pl.BlockSpec((NC, RB, H), lambda i: (0, i, 0)),
            pl.BlockSpec((NC, RB, 1), lambda i: (0, i, 0)),
            pl.BlockSpec((RB, H), lambda i: (i, 0)),
            pl.BlockSpec((1, H), lambda i: (0, 0)),
            pl.BlockSpec((H, 3 * H), lambda i: (0, 0)),
            pl.BlockSpec((H, 3 * H), lambda i: (0, 0)),
            pl.BlockSpec((1, 3 * H), lambda i: (0, 0)),
            pl.BlockSpec((1, 3 * H), lambda i: (0, 0)),
            pl.BlockSpec((H, H), lambda i: (0, 0)),
            pl.BlockSpec((H, 1), lambda i: (0, 0)),
        ],
        out_specs=[
            pl.BlockSpec((RB, H), lambda i: (i, 0)),
            pl.BlockSpec((RB, H), lambda i: (i, 0)),
            pl.BlockSpec((RB, 1), lambda i: (i, 0)),
        ],
        out_shape=[
            jax.ShapeDtypeStruct((N, H), jnp.float32),
            jax.ShapeDtypeStruct((N, H), jnp.float32),
            jax.ShapeDtypeStruct((N, 1), jnp.float32),
        ],
    )


def _stage_e2(N, H, Bn, OUT):
    c00 = (((0,), (0,)), ((), ()))

    def body(x3_ref, xs_ref, amol_ref, bt_ref, mw_ref, md_ref, mb_ref,
             wi_ref, wh_ref, bi_ref, bh_ref, l2_ref, l2b_ref, res_ref):
        bt = bt_ref[...]
        iot = lax.broadcasted_iota(jnp.int32, (N, Bn), 1)
        Mt = (bt == iot).astype(jnp.float32)
        x3 = x3_ref[...]
        out = jnp.maximum(
            lax.dot_general(Mt, x3, c00, preferred_element_type=jnp.float32,
                 precision=lax.Precision.HIGHEST), 0.0)
        xs = xs_ref[...]
        amol = amol_ref[...]
        for _ in range(NUM_TIMESTEPS):
            od = jnp.dot(out, mw_ref[...], preferred_element_type=jnp.float32,
                 precision=lax.Precision.HIGHEST)
            adm = jnp.dot(od, md_ref[...], preferred_element_type=jnp.float32,
                 precision=lax.Precision.HIGHEST)
            a0 = amol + jnp.dot(Mt, adm, preferred_element_type=jnp.float32,
                 precision=lax.Precision.HIGHEST)
            e = jnp.exp(jnp.maximum(a0, 0.01 * a0))
            Zb = lax.dot_general(Mt, e, c00, preferred_element_type=jnp.float32,
                 precision=lax.Precision.HIGHEST)
            Pb = lax.dot_general(Mt, xs * e, c00, preferred_element_type=jnp.float32,
                 precision=lax.Precision.HIGHEST)
            h = _elu(Pb / (Zb + 1e-16) + mb_ref[...])
            out = jnp.maximum(_gru(h, out, wi_ref[...], wh_ref[...],
                                   bi_ref[...], bh_ref[...]), 0.0)
        res_ref[...] = jnp.dot(out, l2_ref[...],
                               preferred_element_type=jnp.float32,
                 precision=lax.Precision.HIGHEST) + l2b_ref[...]

    return pl.pallas_call(
        body,
        out_shape=jax.ShapeDtypeStruct((Bn, OUT), jnp.float32),
        compiler_params=pltpu.CompilerParams(
            vmem_limit_bytes=100 * 1024 * 1024),
    )


def _run(x, edge_index, edge_attr, batch, Bn,
         lin1_W, lin1_b, att_l, att_r, gate_lin1_W, gate_lin2_W, gate_bias,
         gru0_Wi, gru0_Wh, gru0_bi, gru0_bh,
         gat_W, gat_att_src, gat_att_dst, gat_bias,
         gru1_Wi, gru1_Wh, gru1_bi, gru1_bh,
         mol_W, mol_att_src, mol_att_dst, mol_bias,
         mgru_Wi, mgru_Wh, mgru_bi, mgru_bh,
         lin2_W, lin2_b):
    N, IN = x.shape
    H = lin1_W.shape[0]
    E = edge_index.shape[1]
    ED = edge_attr.shape[1]
    OUT = lin2_W.shape[0]

    src = edge_index[0]
    dst = edge_index[1]
    W1a = gate_lin1_W[:, :H]
    W1b = gate_lin1_W[:, H:]

    x1, xa, rdot = _stage_a(N, IN, H)(
        x, lin1_W.T, lin1_b[None, :], W1a.T, att_r[:, None])
    t = _stage_t(E, ED, H)(edge_attr, W1b.T)
    p1, z1 = _edge_gate(N, E, H)(xa, t, rdot.reshape(-1), att_l, src, dst)
    x2, xp, asrc, adst = _stage_c(N, H)(
        p1, z1[:, :, None], x1, gate_lin2_W.T, gate_bias[None, :],
        gru0_Wi.T, gru0_Wh.T, gru0_bi[None, :], gru0_bh[None, :],
        gat_W.T, gat_att_src[:, None], gat_att_dst[:, None])
    p2, z2 = _edge_gat(N, E, H)(xp, asrc.reshape(-1), adst.reshape(-1), src, dst)
    x3, xs, amol = _stage_e1(N, H)(
        p2, z2[:, :, None], x2, gat_bias[None, :],
        gru1_Wi.T, gru1_Wh.T, gru1_bi[None, :], gru1_bh[None, :],
        mol_W.T, mol_att_src[:, None])
    res = _stage_e2(N, H, Bn, OUT)(
        x3, xs, amol, batch[:, None].astype(jnp.int32),
        mol_W.T, mol_att_dst[:, None], mol_bias[None, :],
        mgru_Wi.T, mgru_Wh.T, mgru_bi[None, :], mgru_bh[None, :],
        lin2_W.T, lin2_b[None, :])
    return res


def kernel(x, edge_index, edge_attr, batch,
           lin1_W, lin1_b, att_l, att_r, gate_lin1_W, gate_lin2_W, gate_bias,
           gru0_Wi, gru0_Wh, gru0_bi, gru0_bh,
           gat_W, gat_att_src, gat_att_dst, gat_bias,
           gru1_Wi, gru1_Wh, gru1_bi, gru1_bh,
           mol_W, mol_att_src, mol_att_dst, mol_bias,
           mgru_Wi, mgru_Wh, mgru_bi, mgru_bh,
           lin2_W, lin2_b):
    return _run(x, edge_index, edge_attr, batch, 64,
                lin1_W, lin1_b, att_l, att_r, gate_lin1_W, gate_lin2_W,
                gate_bias, gru0_Wi, gru0_Wh, gru0_bi, gru0_bh,
                gat_W, gat_att_src, gat_att_dst, gat_bias,
                gru1_Wi, gru1_Wh, gru1_bi, gru1_bh,
                mol_W, mol_att_src, mol_att_dst, mol_bias,
                mgru_Wi, mgru_Wh, mgru_bi, mgru_bh,
                lin2_W, lin2_b)
